# Initial kernel scaffold; baseline (speedup 1.0000x reference)
import jax
import jax.numpy as jnp
from jax.experimental import pallas as pl


def kernel(x, edge_index, batch, W1, b1, W2, b2, Wfc, bfc):
    raise NotImplementedError("write your pallas kernel here")


# speedup vs baseline: 13.6014x; 13.6014x over previous
import functools

import jax
import jax.numpy as jnp
from jax import lax
from jax.experimental import pallas as pl
from jax.experimental.pallas import tpu as pltpu
from jax.experimental.pallas import tpu_sc as plsc

N_NODES = 10000
N_EDGES = 320000
D = 128
N_GRAPHS = 64

NC = 2
NS = 16
NW = NC * NS
EPW = N_EDGES // NW
CHUNK = 128
NFULL = EPW // CHUNK
REM = EPW - NFULL * CHUNK
NPAD = 10112
RPT = NPAD // NS


@functools.cache
def _deg_kernel():
    mesh = plsc.VectorSubcoreMesh(core_axis_name="c", subcore_axis_name="s")
    return functools.partial(
        pl.kernel,
        out_type=jax.ShapeDtypeStruct((NC * NPAD, D), jnp.float32),
        mesh=mesh,
        scratch_types=[
            pltpu.VMEM((CHUNK,), jnp.int32),
            pltpu.VMEM((REM,), jnp.int32),
            pltpu.VMEM((CHUNK, D), jnp.float32),
            pltpu.VMEM_SHARED((NPAD, D), jnp.float32),
        ],
    )(_deg_body)


def _deg_body(dst_hbm, zeros_hbm, ones_hbm, out_hbm, dst_v, dst_r, ones_v,
              acc_sh):
    c = lax.axis_index("c")
    s = lax.axis_index("s")
    pltpu.sync_copy(ones_hbm, ones_v)
    pltpu.sync_copy(zeros_hbm.at[pl.ds(s * RPT, RPT)],
                    acc_sh.at[pl.ds(s * RPT, RPT)])
    plsc.subcore_barrier()

    base = (c * NS + s) * EPW

    def body(i, carry):
        pltpu.sync_copy(dst_hbm.at[pl.ds(base + i * CHUNK, CHUNK)], dst_v)
        pltpu.sync_copy(ones_v, acc_sh.at[dst_v], add=True)
        return carry

    lax.fori_loop(0, NFULL, body, 0)
    pltpu.sync_copy(dst_hbm.at[pl.ds(base + NFULL * CHUNK, REM)], dst_r)
    pltpu.sync_copy(ones_v.at[pl.ds(0, REM)], acc_sh.at[dst_r], add=True)

    plsc.subcore_barrier()
    pltpu.sync_copy(acc_sh.at[pl.ds(s * RPT, RPT)],
                    out_hbm.at[pl.ds(c * NPAD + s * RPT, RPT)])


@functools.cache
def _scatter_kernel():
    mesh = plsc.VectorSubcoreMesh(core_axis_name="c", subcore_axis_name="s")
    return functools.partial(
        pl.kernel,
        out_type=jax.ShapeDtypeStruct((NC * NPAD, D), jnp.float32),
        mesh=mesh,
        scratch_types=[
            pltpu.VMEM((CHUNK,), jnp.int32),
            pltpu.VMEM((CHUNK,), jnp.int32),
            pltpu.VMEM((REM,), jnp.int32),
            pltpu.VMEM((REM,), jnp.int32),
            pltpu.VMEM((CHUNK, D), jnp.float32),
            pltpu.VMEM((REM, D), jnp.float32),
            pltpu.VMEM_SHARED((NPAD, D), jnp.float32),
            pltpu.SemaphoreType.DMA,
        ],
    )(_scatter_body)


def _scatter_body(y_hbm, src_hbm, dst_hbm, zeros_hbm, out_hbm,
                  src_v, dst_v, src_r, dst_r, gbuf, gbuf_r, acc_sh, sem):
    c = lax.axis_index("c")
    s = lax.axis_index("s")
    pltpu.sync_copy(zeros_hbm.at[pl.ds(s * RPT, RPT)],
                    acc_sh.at[pl.ds(s * RPT, RPT)])
    plsc.subcore_barrier()

    base = (c * NS + s) * EPW

    def body(i, carry):
        e0 = base + i * CHUNK
        pltpu.sync_copy(src_hbm.at[pl.ds(e0, CHUNK)], src_v)
        pltpu.sync_copy(dst_hbm.at[pl.ds(e0, CHUNK)], dst_v)
        pltpu.async_copy(y_hbm.at[src_v], gbuf, sem).wait()
        pltpu.sync_copy(gbuf, acc_sh.at[dst_v], add=True)
        return carry

    lax.fori_loop(0, NFULL, body, 0)
    e0 = base + NFULL * CHUNK
    pltpu.sync_copy(src_hbm.at[pl.ds(e0, REM)], src_r)
    pltpu.sync_copy(dst_hbm.at[pl.ds(e0, REM)], dst_r)
    pltpu.async_copy(y_hbm.at[src_r], gbuf_r, sem).wait()
    pltpu.sync_copy(gbuf_r, acc_sh.at[dst_r], add=True)

    plsc.subcore_barrier()
    pltpu.sync_copy(acc_sh.at[pl.ds(s * RPT, RPT)],
                    out_hbm.at[pl.ds(c * NPAD + s * RPT, RPT)])


BLK = 200
GRID = N_NODES // BLK


def _p1_body(x_ref, d0_ref, d1_ref, w1_ref, y1_ref, dinv_ref):
    deg = d0_ref[:, 0:1] + d1_ref[:, 0:1] + 1.0
    dinv = lax.rsqrt(deg)
    xw = jnp.dot(x_ref[...], w1_ref[...], preferred_element_type=jnp.float32)
    y1_ref[...] = dinv * xw
    dinv_ref[...] = jnp.broadcast_to(dinv, (BLK, D))


def _p3_body(z0_ref, z1_ref, y1_ref, dinv_ref, b1_ref, w2_ref, y2_ref):
    h = dinv_ref[...] * (z0_ref[...] + z1_ref[...] + y1_ref[...]) + b1_ref[...]
    h = jnp.maximum(h, 0.0)
    y2_ref[...] = dinv_ref[...] * jnp.dot(
        h, w2_ref[...], preferred_element_type=jnp.float32)


def _p5_body(z0_ref, z1_ref, y2_ref, dinv_ref, b2_ref, bb_ref, wfc_ref,
             bfc_ref, out_ref, sums_sc, cnts_sc):
    i = pl.program_id(0)

    @pl.when(i == 0)
    def _():
        sums_sc[...] = jnp.zeros_like(sums_sc)
        cnts_sc[...] = jnp.zeros_like(cnts_sc)

    h = dinv_ref[...] * (z0_ref[...] + z1_ref[...] + y2_ref[...]) + b2_ref[...]
    h = jnp.maximum(h, 0.0)
    gid = lax.broadcasted_iota(jnp.int32, (BLK, N_GRAPHS), 1).astype(jnp.float32)
    p = (bb_ref[...] == gid).astype(jnp.float32)
    dn = (((0,), (0,)), ((), ()))
    sums_sc[...] += lax.dot_general(p, h, dn, preferred_element_type=jnp.float32)
    cnts_sc[...] += lax.dot_general(p, jnp.ones((BLK, D), jnp.float32), dn,
                                    preferred_element_type=jnp.float32)

    @pl.when(i == GRID - 1)
    def _():
        pooled = sums_sc[...] / jnp.maximum(cnts_sc[...], 1.0)
        o = jnp.dot(pooled, wfc_ref[...],
                    preferred_element_type=jnp.float32) + bfc_ref[...]
        out_ref[...] = jnp.maximum(o, 0.0)


def _row_spec():
    return pl.BlockSpec((BLK, D), lambda i: (i, 0))


def _phase1(x, d0, d1, w1):
    return pl.pallas_call(
        _p1_body,
        grid=(GRID,),
        in_specs=[
            _row_spec(),
            _row_spec(),
            _row_spec(),
            pl.BlockSpec((D, D), lambda i: (0, 0)),
        ],
        out_specs=[_row_spec(), _row_spec()],
        out_shape=[jax.ShapeDtypeStruct((N_NODES, D), jnp.float32),
                   jax.ShapeDtypeStruct((N_NODES, D), jnp.float32)],
    )(x, d0, d1, w1)


def _phase3(z0, z1, y1, dinv_b, b1, w2):
    return pl.pallas_call(
        _p3_body,
        grid=(GRID,),
        in_specs=[
            _row_spec(), _row_spec(), _row_spec(), _row_spec(),
            pl.BlockSpec((1, D), lambda i: (0, 0)),
            pl.BlockSpec((D, D), lambda i: (0, 0)),
        ],
        out_specs=_row_spec(),
        out_shape=jax.ShapeDtypeStruct((N_NODES, D), jnp.float32),
    )(z0, z1, y1, dinv_b, b1, w2)


def _phase5(z0, z1, y2, dinv_b, b2, batchb, wfc, bfc):
    return pl.pallas_call(
        _p5_body,
        grid=(GRID,),
        in_specs=[
            _row_spec(), _row_spec(), _row_spec(), _row_spec(),
            pl.BlockSpec((1, D), lambda i: (0, 0)),
            pl.BlockSpec((BLK, N_GRAPHS), lambda i: (i, 0)),
            pl.BlockSpec((D, D), lambda i: (0, 0)),
            pl.BlockSpec((1, D), lambda i: (0, 0)),
        ],
        out_specs=pl.BlockSpec((N_GRAPHS, D), lambda i: (0, 0)),
        out_shape=jax.ShapeDtypeStruct((N_GRAPHS, D), jnp.float32),
        scratch_shapes=[pltpu.VMEM((N_GRAPHS, D), jnp.float32),
                        pltpu.VMEM((N_GRAPHS, D), jnp.float32)],
    )(z0, z1, y2, dinv_b, b2, batchb, wfc, bfc)


def kernel(x, edge_index, batch, W1, b1, W2, b2, Wfc, bfc):
    src = edge_index[0].astype(jnp.int32)
    dst = edge_index[1].astype(jnp.int32)
    batchb = jnp.broadcast_to(
        batch.astype(jnp.float32)[:, None], (N_NODES, N_GRAPHS))
    zeros_d = jnp.zeros((NPAD, D), jnp.float32)
    ones_d = jnp.ones((CHUNK, D), jnp.float32)

    deg_parts = _deg_kernel()(dst, zeros_d, ones_d)
    y1, dinv_b = _phase1(x, deg_parts[:N_NODES], deg_parts[NPAD:NPAD + N_NODES], W1)
    z1 = _scatter_kernel()(y1, src, dst, zeros_d)
    y2 = _phase3(z1[:N_NODES], z1[NPAD:NPAD + N_NODES], y1, dinv_b,
                 b1.reshape(1, D), W2)
    z2 = _scatter_kernel()(y2, src, dst, zeros_d)
    return _phase5(z2[:N_NODES], z2[NPAD:NPAD + N_NODES], y2, dinv_b,
                   b2.reshape(1, D), batchb, Wfc, bfc.reshape(1, D))
